# baseline (speedup 1.0000x reference)
import functools

import jax
import jax.numpy as jnp
from jax import lax
from jax.experimental import pallas as pl
from jax.experimental.pallas import tpu as pltpu

_MT = 256


def _sig(v):
    return 0.5 * jnp.tanh(0.5 * v) + 0.5


def _manual_mm(get_lhs, get_rhs, kt, nt, m_rows, out_dtype):
    entries = m_rows // 4
    addr_of_n = []
    acc_base = [0, 0]
    slot = [0, 0]
    for n in range(nt):
        mxu = n % 2
        addr_of_n.append(acc_base[mxu])
        acc_base[mxu] += entries
        for k in range(kt):
            s = slot[mxu]
            slot[mxu] ^= 1
            pltpu.matmul_push_rhs(get_rhs(k, n), s, mxu)
            pltpu.matmul_acc_lhs(addr_of_n[n], get_lhs(k), mxu,
                                 load_staged_rhs=s)
    return [
        pltpu.matmul_pop(addr_of_n[n], (m_rows, _MT), out_dtype, n % 2)
        for n in range(nt)
    ]


def _stacked_kernel(x_ref, trial_ref,
                    w0p_ref, b0p_ref, wblk0_ref,
                    w1p_ref, b1p_ref, wblk1_ref,
                    w2p_ref, b2p_ref, wblk2_ref,
                    w0a_ref, w0b_ref, b0_ref, midw_ref, midb_ref,
                    wl_ref, bl_ref,
                    out_rnn_ref, out_log_ref,
                    xg, seq0, seq1, *, T, B, H):
    G = 4 * H
    GG = 8 * H
    f32 = jnp.float32
    bf16 = jnp.bfloat16
    M = T * B
    RCH = 256

    def cell0(g):
        i_g = _sig(g[:, 0 * H:1 * H])
        g_g = jnp.tanh(g[:, 2 * H:3 * H])
        o_g = _sig(g[:, 3 * H:4 * H])
        c = i_g * g_g
        h = o_g * jnp.tanh(c)
        return h, c

    def cell_parts(xgate, pif, pgo, c_prev):
        i_g = _sig(xgate[:, 0 * H:1 * H] + pif[:, 0:H])
        f_g = _sig(xgate[:, 1 * H:2 * H] + pif[:, H:2 * H])
        g_g = jnp.tanh(xgate[:, 2 * H:3 * H] + pgo[:, 0:H])
        o_g = _sig(xgate[:, 3 * H:4 * H] + pgo[:, H:2 * H])
        c = f_g * c_prev + i_g * g_g
        h = o_g * jnp.tanh(c)
        return h, c

    def proj_rows(inp_ref, wp_ref, bp_ref, rs, m_rows, n_lo, n_hi, col0):
        kt = wp_ref.shape[0]

        def get_lhs(k):
            return inp_ref[pl.ds(rs, m_rows), k * _MT:(k + 1) * _MT]

        def get_rhs(k, n):
            return wp_ref[k, n_lo + n]

        pops = _manual_mm(get_lhs, get_rhs, kt, n_hi - n_lo, m_rows, f32)
        for j, p in enumerate(pops):
            n = n_lo + j
            xg[pl.ds(rs, m_rows), col0 + j * _MT:col0 + (j + 1) * _MT] = (
                p + bp_ref[:, n * _MT:(n + 1) * _MT])

    def run_layer(inp_ref, wp_ref, bp_ref, wblk_ref, seq_out, l):
        last = seq_out is None

        nt = wp_ref.shape[1]

        def proj_chunk(ch, carry):
            rs = pl.multiple_of(ch * RCH, 16)
            if not last:
                proj_rows(inp_ref, wp_ref, bp_ref, rs, RCH, 0, nt, 0)
            else:
                proj_rows(inp_ref, wp_ref, bp_ref, rs, RCH, 0, nt // 2, 0)
            return carry

        lax.fori_loop(0, M // RCH, proj_chunk, 0)
        if last:
            proj_rows(inp_ref, wp_ref, bp_ref, (T - 1) * B, B, nt // 2, nt, G)


        g0f = xg[0:B, 0:G]
        g0b = xg[(T - 1) * B:T * B, G:GG]
        hf, cf = cell0(g0f)
        hb, cb = cell0(g0b)
        out_rnn_ref[:, l * 2 * H + H:l * 2 * H + 2 * H] = hb
        if not last:
            seq_out[0:B, 0:H] = hf.astype(bf16)
            seq_out[(T - 1) * B:T * B, H:2 * H] = hb.astype(bf16)

        steps_left = T - 1
        u = next(c for c in (9, 7, 3, 1) if steps_left % c == 0)
        n_outer = steps_left // u

        def one_step(t, hf, cf, hb, cb):
            rf = pl.multiple_of(t * B, 16)
            if last:
                hcat = jnp.concatenate(
                    [hf.astype(bf16), hf.astype(bf16)], axis=1)
            else:
                hcat = jnp.concatenate(
                    [hf.astype(bf16), hb.astype(bf16)], axis=1)
            pltpu.matmul_push_rhs(wblk_ref[0, 0], 0, 0)
            pltpu.matmul_acc_lhs(0, hcat, 0, load_staged_rhs=0)
            pltpu.matmul_push_rhs(wblk_ref[0, 1], 0, 1)
            pltpu.matmul_acc_lhs(0, hcat, 1, load_staged_rhs=0)
            if not last:
                pltpu.matmul_push_rhs(wblk_ref[1, 0], 1, 0)
                pltpu.matmul_acc_lhs(64, hcat, 0, load_staged_rhs=1)
                pltpu.matmul_push_rhs(wblk_ref[1, 1], 1, 1)
                pltpu.matmul_acc_lhs(64, hcat, 1, load_staged_rhs=1)
            pif = pltpu.matmul_pop(0, (B, 2 * H), f32, 0)
            pgo = pltpu.matmul_pop(0, (B, 2 * H), f32, 1)
            hf, cf = cell_parts(xg[pl.ds(rf, B), 0:G], pif, pgo, cf)
            if not last:
                rb = pl.multiple_of((T - 1 - t) * B, 16)
                qif = pltpu.matmul_pop(64, (B, 2 * H), f32, 0)
                qgo = pltpu.matmul_pop(64, (B, 2 * H), f32, 1)
                hb, cb = cell_parts(xg[pl.ds(rb, B), G:GG], qif, qgo, cb)
                seq_out[pl.ds(rf, B), 0:H] = hf.astype(bf16)
                seq_out[pl.ds(rb, B), H:2 * H] = hb.astype(bf16)
            return hf, cf, hb, cb

        def outer(k, carry):
            t0 = 1 + k * u
            for uu in range(u):
                carry = one_step(t0 + uu, *carry)
            return carry

        hf, cf, hb, cb = lax.fori_loop(0, n_outer, outer, (hf, cf, hb, cb))
        out_rnn_ref[:, l * 2 * H:l * 2 * H + H] = hf

    run_layer(x_ref, w0p_ref, b0p_ref, wblk0_ref, seq0, 0)
    run_layer(seq0, w1p_ref, b1p_ref, wblk1_ref, seq1, 1)
    run_layer(seq1, w2p_ref, b2p_ref, wblk2_ref, None, 2)

    rnn = out_rnn_ref[...]

    def lhs_rnn(k):
        return rnn[:, k * _MT:(k + 1) * _MT]

    h0_parts = _manual_mm(lhs_rnn, lambda k, n: w0a_ref[k, n],
                          w0a_ref.shape[0], w0a_ref.shape[1], B, f32)
    tr_parts = _manual_mm(lambda k: trial_ref[...],
                          lambda k, n: w0b_ref[k, n],
                          w0b_ref.shape[0], w0b_ref.shape[1], B, f32)
    h0 = jnp.concatenate(
        [jnp.maximum(a + t + b0_ref[:, j * _MT:(j + 1) * _MT], 0.0)
         for j, (a, t) in enumerate(zip(h0_parts, tr_parts))], axis=1)

    def lhs_h0(k):
        return h0[:, k * _MT:(k + 1) * _MT]

    h1_parts = _manual_mm(lhs_h0, lambda k, n: midw_ref[k, n],
                          midw_ref.shape[0], midw_ref.shape[1], B, f32)
    h1 = jnp.concatenate(
        [jnp.maximum(p + midb_ref[:, j * _MT:(j + 1) * _MT], 0.0)
         for j, p in enumerate(h1_parts)], axis=1)

    def lhs_h1(k):
        return h1[:, k * _MT:(k + 1) * _MT]

    (logits,) = _manual_mm(lhs_h1, lambda k, n: wl_ref[k, n],
                           wl_ref.shape[0], wl_ref.shape[1], B, f32)
    n_class = out_log_ref.shape[1]
    out_log_ref[...] = logits[:, 0:n_class] + bl_ref[...]


def _tile_rhs(w, dtype):
    K, N = w.shape
    kp = (-K) % _MT
    np_ = (-N) % _MT
    if kp or np_:
        w = jnp.pad(w, ((0, kp), (0, np_)))
    kt, ntt = (K + kp) // _MT, (N + np_) // _MT
    return w.reshape(kt, _MT, ntt, _MT).transpose(0, 2, 1, 3).astype(dtype)


def _wblk(whh_f, whh_b):
    H = whh_f.shape[0]
    z = jnp.zeros((H, 2 * H), jnp.bfloat16)
    tiles = []
    for w, pos in ((whh_f, 0), (whh_b, 1)):
        row = []
        for m in range(2):
            blk = w[:, m * 2 * H:(m + 1) * 2 * H].astype(jnp.bfloat16)
            parts = [blk, z] if pos == 0 else [z, blk]
            row.append(jnp.concatenate(parts, axis=0))
        tiles.append(jnp.stack(row))
    return jnp.stack(tiles)


def kernel(x, trial_vec,
           rnn0_w_proj, rnn0_b_proj, rnn0_whh_f, rnn0_whh_b,
           rnn1_w_proj, rnn1_b_proj, rnn1_whh_f, rnn1_whh_b,
           rnn2_w_proj, rnn2_b_proj, rnn2_whh_f, rnn2_whh_b,
           fc_w0a, fc_w0b, fc_b0, fc_mid0_w, fc_mid0_b, fc_wl, fc_bl):
    B, T, D = x.shape
    H = rnn0_whh_f.shape[0]
    M = T * B
    n_class = fc_bl.shape[-1]
    n_rnn_feat = fc_w0a.shape[0]

    x2 = jnp.transpose(x.astype(jnp.bfloat16), (1, 0, 2)).reshape(M, D)
    trial = jnp.pad(trial_vec.astype(jnp.float32),
                    ((0, 0), (0, (-trial_vec.shape[1]) % _MT)))
    bf16 = jnp.bfloat16
    f32 = jnp.float32
    w0p_t = _tile_rhs(rnn0_w_proj, bf16)
    w1p_t = _tile_rhs(rnn1_w_proj, bf16)
    w2p_t = _tile_rhs(rnn2_w_proj, bf16)
    wblk0 = _wblk(rnn0_whh_f, rnn0_whh_b)
    wblk1 = _wblk(rnn1_whh_f, rnn1_whh_b)
    wblk2 = _wblk(rnn2_whh_f, rnn2_whh_b)
    w0a_t = _tile_rhs(fc_w0a, f32)
    w0b_t = _tile_rhs(fc_w0b, f32)
    midw_t = _tile_rhs(fc_mid0_w, f32)
    wl_t = _tile_rhs(fc_wl, f32)

    vspec = pl.BlockSpec(memory_space=pltpu.MemorySpace.VMEM)
    out_rnn, out_log = pl.pallas_call(
        functools.partial(_stacked_kernel, T=T, B=B, H=H),
        out_shape=(
            jax.ShapeDtypeStruct((B, n_rnn_feat), jnp.float32),
            jax.ShapeDtypeStruct((B, n_class), jnp.float32),
        ),
        in_specs=[vspec] * 18,
        out_specs=(vspec, vspec),
        scratch_shapes=[
            pltpu.VMEM((M, 8 * H), jnp.float32),
            pltpu.VMEM((M, 2 * H), jnp.bfloat16),
            pltpu.VMEM((M, 2 * H), jnp.bfloat16),
        ],
        compiler_params=pltpu.CompilerParams(
            vmem_limit_bytes=56 * 1024 * 1024,
        ),
    )(x2, trial,
      w0p_t, rnn0_b_proj, wblk0,
      w1p_t, rnn1_b_proj, wblk1,
      w2p_t, rnn2_b_proj, wblk2,
      w0a_t, w0b_t, fc_b0, midw_t, fc_mid0_b, wl_t, fc_bl)
    return out_rnn, out_log


# speedup vs baseline: 1.1363x; 1.1363x over previous
import functools

import jax
import jax.numpy as jnp
from jax import lax
from jax.experimental import pallas as pl
from jax.experimental.pallas import tpu as pltpu

_MT = 256


def _sig(v):
    return 0.5 * jnp.tanh(0.5 * v) + 0.5


def _manual_mm(get_lhs, get_rhs, kt, nt, m_rows, out_dtype, mrb_base=0):
    entries = m_rows // 4
    addr_of_n = []
    acc_base = [mrb_base, mrb_base]
    slot = [0, 0]
    for n in range(nt):
        mxu = n % 2
        addr_of_n.append(acc_base[mxu])
        acc_base[mxu] += entries
        for k in range(kt):
            s = slot[mxu]
            slot[mxu] ^= 1
            pltpu.matmul_push_rhs(get_rhs(k, n), s, mxu)
            pltpu.matmul_acc_lhs(addr_of_n[n], get_lhs(k), mxu,
                                 load_staged_rhs=s)
    return [
        pltpu.matmul_pop(addr_of_n[n], (m_rows, _MT), out_dtype, n % 2)
        for n in range(nt)
    ]


def _stacked_kernel(x_ref, trial_ref,
                    w0p_ref, b0p_ref, wblk0_ref,
                    w1p_ref, b1p_ref, wblk1_ref,
                    w2p_ref, b2p_ref, wblk2_ref,
                    w0a_ref, w0b_ref, b0_ref, midw_ref, midb_ref,
                    wl_ref, bl_ref,
                    out_rnn_ref, out_log_ref,
                    xg, seq0, seq1, *, T, B, H):
    G = 4 * H
    GG = 8 * H
    f32 = jnp.float32
    bf16 = jnp.bfloat16
    M = T * B
    RCH = 256

    def cell0(g):
        i_g = _sig(g[:, 0 * H:1 * H])
        g_g = jnp.tanh(g[:, 2 * H:3 * H])
        o_g = _sig(g[:, 3 * H:4 * H])
        c = i_g * g_g
        h = o_g * jnp.tanh(c)
        return h, c

    def cell_parts(xgate, pif, pgo, c_prev):
        i_g = _sig(xgate[:, 0 * H:1 * H] + pif[:, 0:H])
        f_g = _sig(xgate[:, 1 * H:2 * H] + pif[:, H:2 * H])
        g_g = jnp.tanh(xgate[:, 2 * H:3 * H] + pgo[:, 0:H])
        o_g = _sig(xgate[:, 3 * H:4 * H] + pgo[:, H:2 * H])
        c = f_g * c_prev + i_g * g_g
        h = o_g * jnp.tanh(c)
        return h, c

    def proj_rows(inp_ref, wp_ref, bp_ref, rs, m_rows, n_lo, n_hi, col0,
                  mrb_base=0):
        kt = wp_ref.shape[0]

        def get_lhs(k):
            return inp_ref[pl.ds(rs, m_rows), k * _MT:(k + 1) * _MT]

        def get_rhs(k, n):
            return wp_ref[k, n_lo + n]

        pops = _manual_mm(get_lhs, get_rhs, kt, n_hi - n_lo, m_rows, f32,
                          mrb_base=mrb_base)
        for j, p in enumerate(pops):
            n = n_lo + j
            xg[pl.ds(rs, m_rows), col0 + j * _MT:col0 + (j + 1) * _MT] = (
                p + bp_ref[:, n * _MT:(n + 1) * _MT])

    def run_layer(inp_ref, wp_ref, bp_ref, wblk_ref, seq_out, l):
        last = seq_out is None

        nt = wp_ref.shape[1]

        for ch in range(M // RCH):
            rs = ch * RCH
            base = 128 * (ch % 2)
            if not last:
                proj_rows(inp_ref, wp_ref, bp_ref, rs, RCH, 0, nt, 0, base)
            else:
                proj_rows(inp_ref, wp_ref, bp_ref, rs, RCH, 0, nt // 2, 0,
                          base)
        if last:
            proj_rows(inp_ref, wp_ref, bp_ref, (T - 1) * B, B, nt // 2, nt, G)


        g0f = xg[0:B, 0:G]
        g0b = xg[(T - 1) * B:T * B, G:GG]
        hf, cf = cell0(g0f)
        hb, cb = cell0(g0b)
        out_rnn_ref[:, l * 2 * H + H:l * 2 * H + 2 * H] = hb
        if not last:
            seq_out[0:B, 0:H] = hf.astype(bf16)
            seq_out[(T - 1) * B:T * B, H:2 * H] = hb.astype(bf16)

        steps_left = T - 1
        u = next(c for c in (9, 7, 3, 1) if steps_left % c == 0)
        n_outer = steps_left // u

        def one_step(t, hf, cf, hb, cb):
            rf = pl.multiple_of(t * B, 16)
            if last:
                hcat = jnp.concatenate(
                    [hf.astype(bf16), hf.astype(bf16)], axis=1)
            else:
                hcat = jnp.concatenate(
                    [hf.astype(bf16), hb.astype(bf16)], axis=1)
            pltpu.matmul_push_rhs(wblk_ref[0, 0], 0, 0)
            pltpu.matmul_acc_lhs(0, hcat, 0, load_staged_rhs=0)
            pltpu.matmul_push_rhs(wblk_ref[0, 1], 0, 1)
            pltpu.matmul_acc_lhs(0, hcat, 1, load_staged_rhs=0)
            if not last:
                pltpu.matmul_push_rhs(wblk_ref[1, 0], 1, 0)
                pltpu.matmul_acc_lhs(64, hcat, 0, load_staged_rhs=1)
                pltpu.matmul_push_rhs(wblk_ref[1, 1], 1, 1)
                pltpu.matmul_acc_lhs(64, hcat, 1, load_staged_rhs=1)
            pif = pltpu.matmul_pop(0, (B, 2 * H), f32, 0)
            pgo = pltpu.matmul_pop(0, (B, 2 * H), f32, 1)
            hf, cf = cell_parts(xg[pl.ds(rf, B), 0:G], pif, pgo, cf)
            if not last:
                rb = pl.multiple_of((T - 1 - t) * B, 16)
                qif = pltpu.matmul_pop(64, (B, 2 * H), f32, 0)
                qgo = pltpu.matmul_pop(64, (B, 2 * H), f32, 1)
                hb, cb = cell_parts(xg[pl.ds(rb, B), G:GG], qif, qgo, cb)
                seq_out[pl.ds(rf, B), 0:H] = hf.astype(bf16)
                seq_out[pl.ds(rb, B), H:2 * H] = hb.astype(bf16)
            return hf, cf, hb, cb

        def outer(k, carry):
            t0 = 1 + k * u
            for uu in range(u):
                carry = one_step(t0 + uu, *carry)
            return carry

        hf, cf, hb, cb = lax.fori_loop(0, n_outer, outer, (hf, cf, hb, cb))
        out_rnn_ref[:, l * 2 * H:l * 2 * H + H] = hf

    run_layer(x_ref, w0p_ref, b0p_ref, wblk0_ref, seq0, 0)
    run_layer(seq0, w1p_ref, b1p_ref, wblk1_ref, seq1, 1)
    run_layer(seq1, w2p_ref, b2p_ref, wblk2_ref, None, 2)

    rnn = out_rnn_ref[...]

    def lhs_rnn(k):
        return rnn[:, k * _MT:(k + 1) * _MT]

    h0_parts = _manual_mm(lhs_rnn, lambda k, n: w0a_ref[k, n],
                          w0a_ref.shape[0], w0a_ref.shape[1], B, f32)
    tr_parts = _manual_mm(lambda k: trial_ref[...],
                          lambda k, n: w0b_ref[k, n],
                          w0b_ref.shape[0], w0b_ref.shape[1], B, f32)
    h0 = jnp.concatenate(
        [jnp.maximum(a + t + b0_ref[:, j * _MT:(j + 1) * _MT], 0.0)
         for j, (a, t) in enumerate(zip(h0_parts, tr_parts))], axis=1)

    def lhs_h0(k):
        return h0[:, k * _MT:(k + 1) * _MT]

    h1_parts = _manual_mm(lhs_h0, lambda k, n: midw_ref[k, n],
                          midw_ref.shape[0], midw_ref.shape[1], B, f32)
    h1 = jnp.concatenate(
        [jnp.maximum(p + midb_ref[:, j * _MT:(j + 1) * _MT], 0.0)
         for j, p in enumerate(h1_parts)], axis=1)

    def lhs_h1(k):
        return h1[:, k * _MT:(k + 1) * _MT]

    (logits,) = _manual_mm(lhs_h1, lambda k, n: wl_ref[k, n],
                           wl_ref.shape[0], wl_ref.shape[1], B, f32)
    n_class = out_log_ref.shape[1]
    out_log_ref[...] = logits[:, 0:n_class] + bl_ref[...]


def _tile_rhs(w, dtype):
    K, N = w.shape
    kp = (-K) % _MT
    np_ = (-N) % _MT
    if kp or np_:
        w = jnp.pad(w, ((0, kp), (0, np_)))
    kt, ntt = (K + kp) // _MT, (N + np_) // _MT
    return w.reshape(kt, _MT, ntt, _MT).transpose(0, 2, 1, 3).astype(dtype)


def _wblk(whh_f, whh_b):
    H = whh_f.shape[0]
    z = jnp.zeros((H, 2 * H), jnp.bfloat16)
    tiles = []
    for w, pos in ((whh_f, 0), (whh_b, 1)):
        row = []
        for m in range(2):
            blk = w[:, m * 2 * H:(m + 1) * 2 * H].astype(jnp.bfloat16)
            parts = [blk, z] if pos == 0 else [z, blk]
            row.append(jnp.concatenate(parts, axis=0))
        tiles.append(jnp.stack(row))
    return jnp.stack(tiles)


def kernel(x, trial_vec,
           rnn0_w_proj, rnn0_b_proj, rnn0_whh_f, rnn0_whh_b,
           rnn1_w_proj, rnn1_b_proj, rnn1_whh_f, rnn1_whh_b,
           rnn2_w_proj, rnn2_b_proj, rnn2_whh_f, rnn2_whh_b,
           fc_w0a, fc_w0b, fc_b0, fc_mid0_w, fc_mid0_b, fc_wl, fc_bl):
    B, T, D = x.shape
    H = rnn0_whh_f.shape[0]
    M = T * B
    n_class = fc_bl.shape[-1]
    n_rnn_feat = fc_w0a.shape[0]

    x2 = jnp.transpose(x.astype(jnp.bfloat16), (1, 0, 2)).reshape(M, D)
    trial = jnp.pad(trial_vec.astype(jnp.float32),
                    ((0, 0), (0, (-trial_vec.shape[1]) % _MT)))
    bf16 = jnp.bfloat16
    f32 = jnp.float32
    w0p_t = _tile_rhs(rnn0_w_proj, bf16)
    w1p_t = _tile_rhs(rnn1_w_proj, bf16)
    w2p_t = _tile_rhs(rnn2_w_proj, bf16)
    wblk0 = _wblk(rnn0_whh_f, rnn0_whh_b)
    wblk1 = _wblk(rnn1_whh_f, rnn1_whh_b)
    wblk2 = _wblk(rnn2_whh_f, rnn2_whh_b)
    w0a_t = _tile_rhs(fc_w0a, f32)
    w0b_t = _tile_rhs(fc_w0b, f32)
    midw_t = _tile_rhs(fc_mid0_w, f32)
    wl_t = _tile_rhs(fc_wl, f32)

    vspec = pl.BlockSpec(memory_space=pltpu.MemorySpace.VMEM)
    out_rnn, out_log = pl.pallas_call(
        functools.partial(_stacked_kernel, T=T, B=B, H=H),
        out_shape=(
            jax.ShapeDtypeStruct((B, n_rnn_feat), jnp.float32),
            jax.ShapeDtypeStruct((B, n_class), jnp.float32),
        ),
        in_specs=[vspec] * 18,
        out_specs=(vspec, vspec),
        scratch_shapes=[
            pltpu.VMEM((M, 8 * H), jnp.float32),
            pltpu.VMEM((M, 2 * H), jnp.bfloat16),
            pltpu.VMEM((M, 2 * H), jnp.bfloat16),
        ],
        compiler_params=pltpu.CompilerParams(
            vmem_limit_bytes=56 * 1024 * 1024,
        ),
    )(x2, trial,
      w0p_t, rnn0_b_proj, wblk0,
      w1p_t, rnn1_b_proj, wblk1,
      w2p_t, rnn2_b_proj, wblk2,
      w0a_t, w0b_t, fc_b0, midw_t, fc_mid0_b, wl_t, fc_bl)
    return out_rnn, out_log


# speedup vs baseline: 1.7963x; 1.5808x over previous
import functools

import jax
import jax.numpy as jnp
from jax import lax
from jax.experimental import pallas as pl
from jax.experimental.pallas import tpu as pltpu


def _sig(v):
    return 0.5 * jnp.tanh(0.5 * v) + 0.5


def _stacked_kernel(x_ref, trial_ref,
                    w0p_ref, b0p_ref, whf0_ref, whb0_ref,
                    w1p_ref, b1p_ref, whf1_ref, whb1_ref,
                    w2p_ref, b2p_ref, whf2_ref, whb2_ref,
                    w0a_ref, w0b_ref, b0_ref, midw_ref, midb_ref,
                    wl_ref, bl_ref,
                    out_rnn_ref, out_log_ref,
                    xg, seq0, seq1, *, T, B, H, unroll):
    G = 4 * H
    GG = 8 * H
    f32 = jnp.float32
    bf16 = jnp.bfloat16

    def cell(g, c_prev):
        i_g = _sig(g[:, 0 * H:1 * H])
        f_g = _sig(g[:, 1 * H:2 * H])
        g_g = jnp.tanh(g[:, 2 * H:3 * H])
        o_g = _sig(g[:, 3 * H:4 * H])
        c = f_g * c_prev + i_g * g_g
        h = o_g * jnp.tanh(c)
        return h, c

    def cell0(g):
        i_g = _sig(g[:, 0 * H:1 * H])
        g_g = jnp.tanh(g[:, 2 * H:3 * H])
        o_g = _sig(g[:, 3 * H:4 * H])
        c = i_g * g_g
        h = o_g * jnp.tanh(c)
        return h, c

    def run_layer(inp_ref, wp_ref, bp_ref, whf_ref, whb_ref, seq_out, l):
        last = seq_out is None

        if not last:
            xg[...] = (
                jnp.dot(inp_ref[...], wp_ref[...],
                        preferred_element_type=f32) + bp_ref[...])
        else:
            xg[:, 0:G] = (
                jnp.dot(inp_ref[...], wp_ref[:, 0:G],
                        preferred_element_type=f32) + bp_ref[:, 0:G])
            xg[(T - 1) * B:T * B, G:GG] = (
                jnp.dot(inp_ref[(T - 1) * B:T * B, :], wp_ref[:, G:GG],
                        preferred_element_type=f32) + bp_ref[:, G:GG])

        g0f = xg[0:B, 0:G]
        g0b = xg[(T - 1) * B:T * B, G:GG]
        hf, cf = cell0(g0f)
        hb, cb = cell0(g0b)
        out_rnn_ref[:, l * 2 * H + H:l * 2 * H + 2 * H] = hb
        if not last:
            seq_out[0:B, 0:H] = hf.astype(bf16)
            seq_out[(T - 1) * B:T * B, H:2 * H] = hb.astype(bf16)

        steps_left = T - 1
        u = next(c for c in (unroll, 9, 7, 3, 1) if steps_left % c == 0)
        n_outer = steps_left // u

        def one_step(t, hf, cf, hb, cb):
            rf = pl.multiple_of(t * B, 16)
            gf = xg[pl.ds(rf, B), 0:G] + jnp.dot(
                hf.astype(bf16), whf_ref[...], preferred_element_type=f32)
            hf, cf = cell(gf, cf)
            if not last:
                rb = pl.multiple_of((T - 1 - t) * B, 16)
                gb = xg[pl.ds(rb, B), G:GG] + jnp.dot(
                    hb.astype(bf16), whb_ref[...], preferred_element_type=f32)
                hb, cb = cell(gb, cb)
                seq_out[pl.ds(rf, B), 0:H] = hf.astype(bf16)
                seq_out[pl.ds(rb, B), H:2 * H] = hb.astype(bf16)
            return hf, cf, hb, cb

        if n_outer == 1:
            carry = (hf, cf, hb, cb)
            for t in range(1, T):
                carry = one_step(t, *carry)
            hf, cf, hb, cb = carry
        else:
            def outer(k, carry):
                t0 = 1 + k * u
                for uu in range(u):
                    carry = one_step(t0 + uu, *carry)
                return carry

            hf, cf, hb, cb = lax.fori_loop(0, n_outer, outer,
                                           (hf, cf, hb, cb))
        out_rnn_ref[:, l * 2 * H:l * 2 * H + H] = hf

    run_layer(x_ref, w0p_ref, b0p_ref, whf0_ref, whb0_ref, seq0, 0)
    run_layer(seq0, w1p_ref, b1p_ref, whf1_ref, whb1_ref, seq1, 1)
    run_layer(seq1, w2p_ref, b2p_ref, whf2_ref, whb2_ref, None, 2)

    rnn = out_rnn_ref[...]
    h0 = jnp.maximum(
        jnp.dot(rnn, w0a_ref[...], preferred_element_type=f32)
        + jnp.dot(trial_ref[...], w0b_ref[...], preferred_element_type=f32)
        + b0_ref[...], 0.0)
    h1 = jnp.maximum(
        jnp.dot(h0, midw_ref[...], preferred_element_type=f32)
        + midb_ref[...], 0.0)
    out_log_ref[...] = (
        jnp.dot(h1, wl_ref[...], preferred_element_type=f32) + bl_ref[...])


def kernel(x, trial_vec,
           rnn0_w_proj, rnn0_b_proj, rnn0_whh_f, rnn0_whh_b,
           rnn1_w_proj, rnn1_b_proj, rnn1_whh_f, rnn1_whh_b,
           rnn2_w_proj, rnn2_b_proj, rnn2_whh_f, rnn2_whh_b,
           fc_w0a, fc_w0b, fc_b0, fc_mid0_w, fc_mid0_b, fc_wl, fc_bl):
    B, T, D = x.shape
    H = rnn0_whh_f.shape[0]
    M = T * B
    n_class = fc_bl.shape[-1]
    n_rnn_feat = fc_w0a.shape[0]

    x2 = jnp.transpose(x.astype(jnp.bfloat16), (1, 0, 2)).reshape(M, D)
    trial = trial_vec.astype(jnp.float32)

    vspec = pl.BlockSpec(memory_space=pltpu.MemorySpace.VMEM)
    out_rnn, out_log = pl.pallas_call(
        functools.partial(_stacked_kernel, T=T, B=B, H=H, unroll=63),
        out_shape=(
            jax.ShapeDtypeStruct((B, n_rnn_feat), jnp.float32),
            jax.ShapeDtypeStruct((B, n_class), jnp.float32),
        ),
        in_specs=[vspec] * 21,
        out_specs=(vspec, vspec),
        scratch_shapes=[
            pltpu.VMEM((M, 8 * H), jnp.float32),
            pltpu.VMEM((M, 2 * H), jnp.bfloat16),
            pltpu.VMEM((M, 2 * H), jnp.bfloat16),
        ],
        compiler_params=pltpu.CompilerParams(
            vmem_limit_bytes=56 * 1024 * 1024,
        ),
    )(x2, trial,
      rnn0_w_proj, rnn0_b_proj, rnn0_whh_f, rnn0_whh_b,
      rnn1_w_proj, rnn1_b_proj, rnn1_whh_f, rnn1_whh_b,
      rnn2_w_proj, rnn2_b_proj, rnn2_whh_f, rnn2_whh_b,
      fc_w0a, fc_w0b, fc_b0, fc_mid0_w, fc_mid0_b, fc_wl, fc_bl)
    return out_rnn, out_log
